# baseline (speedup 1.0000x reference)
import functools

import jax
import jax.numpy as jnp
from jax import lax
from jax.experimental import pallas as pl
from jax.experimental.pallas import tpu as pltpu
from jax.experimental.pallas import tpu_sc as plsc

_NUM_CLASSES = 10
_ORDERED_P = 0.5
_NT = _NUM_CLASSES - 1

_NC, _NS, _L = 2, 16, 16
_NW = _NC * _NS
_CR = 25
_NBUF = 5


def _class_table(num_cols):
    key = jax.random.key(42)
    kr, kv, kp = jax.random.split(key, 3)
    randomized = jax.random.uniform(kr, (num_cols,)) > _ORDERED_P
    reverse = jax.random.uniform(kv, (num_cols,)) > 0.5
    perm = jax.random.permutation(kp, _NUM_CLASSES).astype(jnp.int32)
    c = jnp.arange(_NUM_CLASSES, dtype=jnp.int32)
    m = jnp.where(randomized[:, None], perm[None, :], c[None, :])
    m = jnp.where(reverse[:, None], _NUM_CLASSES - 1 - m, m)
    return m


def _sc_call(t_sc, b):
    cw = _CR * b
    n_chunks = t_sc // (_NW * _CR)
    rpw = t_sc // _NW
    ngrp = b // _L
    ur = 4

    mesh = plsc.VectorSubcoreMesh(core_axis_name="c", subcore_axis_name="s")

    @functools.partial(
        pl.kernel,
        mesh=mesh,
        out_type=jax.ShapeDtypeStruct((t_sc * b,), jnp.int32),
        scratch_types=[
            pltpu.VMEM((2 * cw,), jnp.float32),
            pltpu.VMEM((2 * cw,), jnp.int32),
            pltpu.VMEM((_NT * _L,), jnp.float32),
            pltpu.VMEM((ngrp * _NT * _L,), jnp.int32),
            pltpu.VMEM((b,), jnp.int32),
            pltpu.SemaphoreType.DMA((2,)),
            pltpu.SemaphoreType.DMA((2,)),
        ],
    )
    def k(x_hbm, s_hbm, d_hbm, l0_hbm, out_hbm, xb, ob, s_v, d_v, l0_v, isem, osem):
        wid = lax.axis_index("s") * _NC + lax.axis_index("c")
        pltpu.sync_copy(s_hbm, s_v)
        pltpu.sync_copy(d_hbm, d_v)
        pltpu.sync_copy(l0_hbm, l0_v)
        base = wid * rpw * b

        def in_dma(ci):
            slot = lax.rem(ci, 2)
            return pltpu.make_async_copy(
                x_hbm.at[pl.ds(base + ci * cw, cw)],
                xb.at[pl.ds(slot * cw, cw)],
                isem.at[slot],
            )

        def out_dma(ci):
            slot = lax.rem(ci, 2)
            return pltpu.make_async_copy(
                ob.at[pl.ds(slot * cw, cw)],
                out_hbm.at[pl.ds(base + ci * cw, cw)],
                osem.at[slot],
            )

        in_dma(0).start()
        in_dma(1).start()

        def chunk_body(ci, _):
            off = lax.rem(ci, 2) * cw
            in_dma(ci).wait()

            @pl.when(ci >= 2)
            def _():
                out_dma(ci - 2).wait()

            for g in range(ngrp):
                l0g = l0_v[pl.ds(g * _L, _L)]
                dg = [d_v[pl.ds((g * _NT + i) * _L, _L)] for i in range(_NT)]
                ss = [s_v[pl.ds(i * _L, _L)] for i in range(_NT)]

                def row_body(r, _):
                    p0 = off + r * (ur * b) + g * _L
                    xv = [xb[pl.ds(p0 + rr * b, _L)] for rr in range(ur)]
                    acc = [l0g] * ur
                    for i in range(_NT):
                        for rr in range(ur):
                            acc[rr] = jnp.where(
                                xv[rr] > ss[i], acc[rr] + dg[i], acc[rr]
                            )
                    for rr in range(ur):
                        ob[pl.ds(p0 + rr * b, _L)] = acc[rr]
                    return 0

                lax.fori_loop(0, _CR // ur, row_body, 0)
                p0 = off + (_CR // ur * ur) * b + g * _L
                xv = xb[pl.ds(p0, _L)]
                acc = l0g
                for i in range(_NT):
                    acc = jnp.where(xv > ss[i], acc + dg[i], acc)
                ob[pl.ds(p0, _L)] = acc

            out_dma(ci).start()

            @pl.when(ci + 2 < n_chunks)
            def _():
                in_dma(ci + 2).start()

            return 0

        lax.fori_loop(0, n_chunks, chunk_body, 0)
        out_dma(n_chunks - 2).wait()
        out_dma(n_chunks - 1).wait()

    return k


def kernel(x, thresholds):
    t, b = x.shape
    m = _class_table(b)
    s_sorted = jnp.sort(thresholds)
    d = (m[:, 1:] - m[:, :-1]).T
    l0 = m[:, 0]

    s16 = jnp.broadcast_to(s_sorted[:, None], (_NT, _L)).reshape(-1)
    d_sc = d.reshape(_NT, b // _L, _L).transpose(1, 0, 2).reshape(-1)

    out_flat = _sc_call(t, b)(x.reshape(-1), s16, d_sc, l0)
    return out_flat.reshape(t, b)


# speedup vs baseline: 2.2378x; 2.2378x over previous
import functools

import jax
import jax.numpy as jnp
from jax import lax
from jax.experimental import pallas as pl
from jax.experimental.pallas import tpu as pltpu
from jax.experimental.pallas import tpu_sc as plsc

_NUM_CLASSES = 10
_ORDERED_P = 0.5
_NT = _NUM_CLASSES - 1

_NC, _NS, _L = 2, 16, 16
_NW = _NC * _NS
_CR = 25
_NBUF = 5

_T_SC = 20000
_TC_BLOCK = 10000


def _class_table(num_cols):
    key = jax.random.key(42)
    kr, kv, kp = jax.random.split(key, 3)
    randomized = jax.random.uniform(kr, (num_cols,)) > _ORDERED_P
    reverse = jax.random.uniform(kv, (num_cols,)) > 0.5
    perm = jax.random.permutation(kp, _NUM_CLASSES).astype(jnp.int32)
    c = jnp.arange(_NUM_CLASSES, dtype=jnp.int32)
    m = jnp.where(randomized[:, None], perm[None, :], c[None, :])
    m = jnp.where(reverse[:, None], _NUM_CLASSES - 1 - m, m)
    return m


def _tree_pass(xv, ss, dg, l0g):
    terms = [jnp.where(xv > ss[i], dg[i], 0) for i in range(_NT)]
    terms.append(l0g)
    while len(terms) > 1:
        terms = [
            terms[j] + terms[j + 1] if j + 1 < len(terms) else terms[j]
            for j in range(0, len(terms), 2)
        ]
    return terms[0]


def _sc_call(t_sc, b):
    cw = _CR * b
    n_chunks = t_sc // (_NW * _CR)
    n_outer = n_chunks // _NBUF
    rpw = t_sc // _NW
    ngrp = b // _L

    mesh = plsc.VectorSubcoreMesh(core_axis_name="c", subcore_axis_name="s")

    @functools.partial(
        pl.kernel,
        mesh=mesh,
        out_type=jax.ShapeDtypeStruct((t_sc * b,), jnp.int32),
        scratch_types=[pltpu.VMEM((cw,), jnp.float32)] * _NBUF
        + [pltpu.VMEM((cw,), jnp.int32)] * _NBUF
        + [
            pltpu.VMEM((_NT * _L,), jnp.float32),
            pltpu.VMEM((ngrp * _NT * _L,), jnp.int32),
            pltpu.VMEM((b,), jnp.int32),
        ]
        + [pltpu.SemaphoreType.DMA] * (2 * _NBUF),
    )
    def k(x_hbm, s_hbm, d_hbm, l0_hbm, out_hbm, *refs):
        x_v = refs[:_NBUF]
        o_v = refs[_NBUF : 2 * _NBUF]
        s_v, d_v, l0_v = refs[2 * _NBUF : 2 * _NBUF + 3]
        sems = refs[2 * _NBUF + 3 :]
        in_sems, out_sems = sems[:_NBUF], sems[_NBUF:]
        wid = lax.axis_index("s") * _NC + lax.axis_index("c")
        pltpu.sync_copy(s_hbm, s_v)
        pltpu.sync_copy(d_hbm, d_v)
        pltpu.sync_copy(l0_hbm, l0_v)
        base = wid * rpw * b

        def in_dma(ci, slot):
            return pltpu.make_async_copy(
                x_hbm.at[pl.ds(base + ci * cw, cw)], x_v[slot], in_sems[slot]
            )

        def out_dma(ci, slot):
            return pltpu.make_async_copy(
                o_v[slot], out_hbm.at[pl.ds(base + ci * cw, cw)], out_sems[slot]
            )

        for slot in range(_NBUF):
            in_dma(slot, slot).start()

        def compute_chunk(slot):
            xs, os = x_v[slot], o_v[slot]
            for g in range(ngrp):
                l0g = l0_v[pl.ds(g * _L, _L)]
                dg = [d_v[pl.ds((g * _NT + i) * _L, _L)] for i in range(_NT)]
                ss = [s_v[pl.ds(i * _L, _L)] for i in range(_NT)]

                def row_body(r, _):
                    p = r * b + g * _L
                    os[pl.ds(p, _L)] = _tree_pass(xs[pl.ds(p, _L)], ss, dg, l0g)
                    return 0

                lax.fori_loop(0, _CR, row_body, 0)

        def outer_body(j, _):
            for slot in range(_NBUF):
                ci = j * _NBUF + slot
                in_dma(ci, slot).wait()

                @pl.when(j > 0)
                def _():
                    out_dma(ci - _NBUF, slot).wait()

                compute_chunk(slot)
                out_dma(ci, slot).start()

                @pl.when(ci + _NBUF < n_chunks)
                def _():
                    in_dma(ci + _NBUF, slot).start()

            return 0

        lax.fori_loop(0, n_outer, outer_body, 0)
        for slot in range(_NBUF):
            out_dma(n_chunks - _NBUF + slot, slot).wait()

    return k


def _tc_body(x_ref, s_ref, d_ref, l0_ref, o_ref):
    x = x_ref[...]
    acc = jnp.broadcast_to(l0_ref[...], x.shape)
    for i in range(_NT):
        acc = jnp.where(x > s_ref[i : i + 1, :], acc + d_ref[i : i + 1, :], acc)
    o_ref[...] = acc


def _tc_call(t_tc, b, x_head, s_rows, d_rows, l0_row):
    grid = t_tc // _TC_BLOCK
    return pl.pallas_call(
        _tc_body,
        grid=(grid,),
        in_specs=[
            pl.BlockSpec((_TC_BLOCK, b), lambda i: (i, 0)),
            pl.BlockSpec((_NT, b), lambda i: (0, 0)),
            pl.BlockSpec((_NT, b), lambda i: (0, 0)),
            pl.BlockSpec((1, b), lambda i: (0, 0)),
        ],
        out_specs=pl.BlockSpec((_TC_BLOCK, b), lambda i: (i, 0)),
        out_shape=jax.ShapeDtypeStruct((t_tc, b), jnp.int32),
    )(x_head, s_rows, d_rows, l0_row)


def kernel(x, thresholds):
    t, b = x.shape
    m = _class_table(b)
    s_sorted = jnp.sort(thresholds)
    d = (m[:, 1:] - m[:, :-1]).T
    l0 = m[:, 0]

    s16 = jnp.broadcast_to(s_sorted[:, None], (_NT, _L)).reshape(-1)
    d_sc = d.reshape(_NT, b // _L, _L).transpose(1, 0, 2).reshape(-1)

    t_tc = t - _T_SC
    sc_out = _sc_call(_T_SC, b)(x[t_tc:].reshape(-1), s16, d_sc, l0)

    s_rows = jnp.broadcast_to(s_sorted[:, None], (_NT, b))
    tc_out = _tc_call(t_tc, b, x[:t_tc], s_rows, d, l0[None, :])

    return jnp.concatenate([tc_out, sc_out.reshape(_T_SC, b)], axis=0)


# speedup vs baseline: 5.5830x; 2.4949x over previous
import jax
import jax.numpy as jnp
from jax.experimental import pallas as pl

_NUM_CLASSES = 10
_ORDERED_P = 0.5
_ROWS_PER_BLOCK = 10000


def _class_table(num_cols):
    key = jax.random.key(42)
    kr, kv, kp = jax.random.split(key, 3)
    randomized = jax.random.uniform(kr, (num_cols,)) > _ORDERED_P
    reverse = jax.random.uniform(kv, (num_cols,)) > 0.5
    perm = jax.random.permutation(kp, _NUM_CLASSES).astype(jnp.int32)
    c = jnp.arange(_NUM_CLASSES, dtype=jnp.int32)
    m = jnp.where(randomized[:, None], perm[None, :], c[None, :])
    m = jnp.where(reverse[:, None], _NUM_CLASSES - 1 - m, m)
    return m


def _body(x_ref, s_ref, d_ref, l0_ref, o_ref):
    x = x_ref[...]
    acc = jnp.broadcast_to(l0_ref[...], x.shape)
    for i in range(_NUM_CLASSES - 1):
        acc = jnp.where(x > s_ref[i : i + 1, :], acc + d_ref[i : i + 1, :], acc)
    o_ref[...] = acc


def kernel(x, thresholds):
    t, b = x.shape
    m = _class_table(b)
    s_sorted = jnp.sort(thresholds)
    s_rows = jnp.broadcast_to(s_sorted[:, None], (_NUM_CLASSES - 1, b))
    d_rows = (m[:, 1:] - m[:, :-1]).T
    l0_row = m[:, 0][None, :]

    grid = t // _ROWS_PER_BLOCK
    return pl.pallas_call(
        _body,
        grid=(grid,),
        in_specs=[
            pl.BlockSpec((_ROWS_PER_BLOCK, b), lambda i: (i, 0)),
            pl.BlockSpec((_NUM_CLASSES - 1, b), lambda i: (0, 0)),
            pl.BlockSpec((_NUM_CLASSES - 1, b), lambda i: (0, 0)),
            pl.BlockSpec((1, b), lambda i: (0, 0)),
        ],
        out_specs=pl.BlockSpec((_ROWS_PER_BLOCK, b), lambda i: (i, 0)),
        out_shape=jax.ShapeDtypeStruct((t, b), jnp.int32),
    )(x, s_rows, d_rows, l0_row)


# speedup vs baseline: 6.7885x; 1.2159x over previous
import jax
import jax.numpy as jnp
from jax.experimental import pallas as pl

_NUM_CLASSES = 10
_ORDERED_P = 0.5
_ROWS_PER_BLOCK = 10000


def _class_table(num_cols):
    key = jax.random.key(42)
    kr, kv, kp = jax.random.split(key, 3)
    randomized = jax.random.uniform(kr, (num_cols,)) > _ORDERED_P
    reverse = jax.random.uniform(kv, (num_cols,)) > 0.5
    perm = jax.random.permutation(kp, _NUM_CLASSES).astype(jnp.int32)
    c = jnp.arange(_NUM_CLASSES, dtype=jnp.int32)
    m = jnp.where(randomized[:, None], perm[None, :], c[None, :])
    m = jnp.where(reverse[:, None], _NUM_CLASSES - 1 - m, m)
    return m


def _body(x_ref, s_ref, v_ref, o_ref):
    x = x_ref[...]

    def s(i):
        return s_ref[i : i + 1, :]

    def v(k):
        return v_ref[k : k + 1, :]

    m4 = x > s(4)
    pb = jnp.where(m4, s(6), s(1))
    mb = x > pb
    pc = jnp.where(m4, jnp.where(mb, s(7), s(5)), jnp.where(mb, s(2), s(0)))
    mc = x > pc
    pd = jnp.where(m4, s(8), s(3))
    md = x > pd
    t1 = jnp.where(md, v(9), v(8))
    t2 = jnp.where(md, v(4), v(3))
    u1 = jnp.where(mc, t1, v(7))
    u2 = jnp.where(mc, v(6), v(5))
    u3 = jnp.where(mc, t2, v(2))
    u4 = jnp.where(mc, v(1), v(0))
    w1 = jnp.where(mb, u1, u2)
    w2 = jnp.where(mb, u3, u4)
    o_ref[...] = jnp.where(m4, w1, w2)


def kernel(x, thresholds):
    t, b = x.shape
    m = _class_table(b)
    s_sorted = jnp.sort(thresholds)
    s_rows = jnp.broadcast_to(s_sorted[:, None], (_NUM_CLASSES - 1, b))
    v_rows = m.T

    grid = t // _ROWS_PER_BLOCK
    return pl.pallas_call(
        _body,
        grid=(grid,),
        in_specs=[
            pl.BlockSpec((_ROWS_PER_BLOCK, b), lambda i: (i, 0)),
            pl.BlockSpec((_NUM_CLASSES - 1, b), lambda i: (0, 0)),
            pl.BlockSpec((_NUM_CLASSES, b), lambda i: (0, 0)),
        ],
        out_specs=pl.BlockSpec((_ROWS_PER_BLOCK, b), lambda i: (i, 0)),
        out_shape=jax.ShapeDtypeStruct((t, b), jnp.int32),
    )(x, s_rows, v_rows)
